# baseline (speedup 1.0000x reference)
import functools

import jax
import jax.numpy as jnp
from jax import lax
from jax.experimental import pallas as pl
from jax.experimental.pallas import tpu as pltpu
from jax.experimental.pallas import tpu_sc as plsc

VOCAB = 100000
WDIM = 128
MODEL_DIM = 128
HID = MODEL_DIM // 2
B = 256
L = 128
DEPTH = 64
NSTEP = DEPTH - 1
BSZ = 2 * B
SLOTS = 2 * L

NW = 32
TOK = BSZ * L
TOK_PER_W = TOK // NW
CHUNK = 128
NCHUNK = TOK_PER_W // CHUNK

E = 128
G = BSZ // E


def _sc_embed_gather(table, idx):
    mesh = plsc.VectorSubcoreMesh(core_axis_name="c", subcore_axis_name="s")

    @functools.partial(
        pl.kernel,
        out_type=jax.ShapeDtypeStruct((TOK, WDIM), jnp.float32),
        mesh=mesh,
        scratch_types=[
            pltpu.VMEM((NCHUNK, CHUNK), jnp.int32),
            pltpu.VMEM((CHUNK, WDIM), jnp.float32),
            pltpu.VMEM((CHUNK, WDIM), jnp.float32),
            pltpu.SemaphoreType.DMA,
            pltpu.SemaphoreType.DMA,
        ],
    )
    def body(table_hbm, idx_hbm, out_hbm, idx_v, rows0, rows1, sem0, sem1):
        wid = lax.axis_index("s") * 2 + lax.axis_index("c")
        base = wid * TOK_PER_W
        pltpu.sync_copy(idx_hbm.at[wid], idx_v)
        bufs = (rows0, rows1)
        sems = (sem0, sem1)
        copies = [None, None]
        copies[0] = pltpu.async_copy(table_hbm.at[idx_v.at[0]], rows0, sem0)
        for j in range(NCHUNK):
            p = j % 2
            copies[p].wait()
            if j + 1 < NCHUNK:
                q = (j + 1) % 2
                copies[q] = pltpu.async_copy(
                    table_hbm.at[idx_v.at[j + 1]], bufs[q], sems[q])
            pltpu.sync_copy(bufs[p], out_hbm.at[pl.ds(base + j * CHUNK, CHUNK)])

    return body(table, idx)


E2 = E // 2


def _tc_tree_body(emb_ref, l_ref, r_ref, w_ref, wsh_ref, we_ref, be_ref,
                  wc_ref, bc_ref, out_ref, buf_ref,
                  lfa_ref, rta_ref, sta_ref, lfb_ref, rtb_ref, stb_ref):
    proj = jnp.dot(emb_ref[...], we_ref[...],
                   preferred_element_type=jnp.float32) + be_ref[...]
    buf_ref[:, :L, :] = proj.reshape(E, L, MODEL_DIM)
    buf_ref[:, L:, :] = jnp.zeros((E, SLOTS - L, MODEL_DIM), jnp.float32)
    stb_ref[...] = jnp.zeros((E2, MODEL_DIM), jnp.float32)

    def lstm(lf, rt):
        hl, cl = lf[:, :HID], lf[:, HID:]
        hr, cr = rt[:, :HID], rt[:, HID:]
        gates = (jnp.dot(hl, wc_ref[:HID, :], preferred_element_type=jnp.float32)
                 + jnp.dot(hr, wc_ref[HID:, :], preferred_element_type=jnp.float32)
                 + bc_ref[...])
        i_g = gates[:, 0 * HID:1 * HID]
        f_l = gates[:, 1 * HID:2 * HID]
        f_r = gates[:, 2 * HID:3 * HID]
        o_g = gates[:, 3 * HID:4 * HID]
        g_g = gates[:, 4 * HID:5 * HID]
        c = (jax.nn.sigmoid(f_l) * cl + jax.nn.sigmoid(f_r) * cr
             + jax.nn.sigmoid(i_g) * jnp.tanh(g_g))
        h = jax.nn.sigmoid(o_g) * jnp.tanh(c)
        return jnp.concatenate([h, c], axis=1)

    for e in range(E2):
        lfa_ref[e, :] = buf_ref[e, l_ref[0, e], :]
        rta_ref[e, :] = buf_ref[e, r_ref[0, e], :]

    def step(i, carry):
        for e in range(E2):
            buf_ref[E2 + e, wsh_ref[i, E2 + e], :] = stb_ref[e, :]
        for e in range(E2):
            lfb_ref[e, :] = buf_ref[E2 + e, l_ref[i, E2 + e], :]
            rtb_ref[e, :] = buf_ref[E2 + e, r_ref[i, E2 + e], :]
        sta_ref[...] = lstm(lfa_ref[...], rta_ref[...])
        stb_ref[...] = lstm(lfb_ref[...], rtb_ref[...])
        for e in range(E2):
            buf_ref[e, w_ref[i, e], :] = sta_ref[e, :]
        for e in range(E2):
            lfa_ref[e, :] = buf_ref[e, l_ref[i + 1, e], :]
            rta_ref[e, :] = buf_ref[e, r_ref[i + 1, e], :]
        return carry

    lax.fori_loop(0, NSTEP, step, 0, unroll=3)
    out_ref[:E2, :] = sta_ref[...]
    out_ref[E2:, :] = stb_ref[...]


def _tc_tree(emb, l_adj, r_adj, w_adj, w_shift, W_enc, b_enc, W_comp, b_comp):
    return pl.pallas_call(
        _tc_tree_body,
        grid=(G,),
        in_specs=[
            pl.BlockSpec((E * L, WDIM), lambda g: (g, 0)),
            pl.BlockSpec((DEPTH, E), lambda g: (0, g), memory_space=pltpu.SMEM),
            pl.BlockSpec((DEPTH, E), lambda g: (0, g), memory_space=pltpu.SMEM),
            pl.BlockSpec((DEPTH, E), lambda g: (0, g), memory_space=pltpu.SMEM),
            pl.BlockSpec((DEPTH, E), lambda g: (0, g), memory_space=pltpu.SMEM),
            pl.BlockSpec((WDIM, MODEL_DIM), lambda g: (0, 0)),
            pl.BlockSpec((1, MODEL_DIM), lambda g: (0, 0)),
            pl.BlockSpec((2 * HID, 5 * HID), lambda g: (0, 0)),
            pl.BlockSpec((1, 5 * HID), lambda g: (0, 0)),
        ],
        out_specs=pl.BlockSpec((E, MODEL_DIM), lambda g: (g, 0)),
        out_shape=jax.ShapeDtypeStruct((BSZ, MODEL_DIM), jnp.float32),
        scratch_shapes=[
            pltpu.VMEM((E, SLOTS, MODEL_DIM), jnp.float32),
            pltpu.VMEM((E2, MODEL_DIM), jnp.float32),
            pltpu.VMEM((E2, MODEL_DIM), jnp.float32),
            pltpu.VMEM((E2, MODEL_DIM), jnp.float32),
            pltpu.VMEM((E2, MODEL_DIM), jnp.float32),
            pltpu.VMEM((E2, MODEL_DIM), jnp.float32),
            pltpu.VMEM((E2, MODEL_DIM), jnp.float32),
        ],
    )(emb, l_adj, r_adj, w_adj, w_shift, W_enc, b_enc, W_comp, b_comp)


@jax.jit
def _run(sentences, left_nodes, right_nodes, write_nodes, embed_table,
         W_enc, b_enc, W_comp, b_comp):
    x = jnp.concatenate([sentences[:, :, 0], sentences[:, :, 1]], axis=0)
    idx = x.reshape(NW, NCHUNK, CHUNK)

    l = jnp.concatenate([left_nodes[:, :, 0], left_nodes[:, :, 1]], axis=0)
    r = jnp.concatenate([right_nodes[:, :, 0], right_nodes[:, :, 1]], axis=0)
    w = jnp.concatenate([write_nodes[:, :, 0], write_nodes[:, :, 1]], axis=0)
    l = l - (l >= 200).astype(l.dtype) * (200 - DEPTH)
    r = r - (r >= 200).astype(r.dtype) * (200 - DEPTH)
    w = w - (w >= 201).astype(w.dtype) * (201 - DEPTH)
    w = w + (w <= 0).astype(w.dtype) * (2 * DEPTH)
    w_shift = jnp.concatenate(
        [jnp.full((BSZ, 1), SLOTS - 1, w.dtype), w[:, :-1]], axis=1)
    l, r, w, w_shift = l.T, r.T, w.T, w_shift.T

    emb = _sc_embed_gather(embed_table, idx)
    return _tc_tree(emb, l, r, w, w_shift, W_enc, b_enc.reshape(1, MODEL_DIM),
                    W_comp, b_comp.reshape(1, 5 * HID))


def kernel(sentences, left_nodes, right_nodes, write_nodes, embed_table,
           W_enc, b_enc, W_comp, b_comp):
    return _run(sentences, left_nodes, right_nodes, write_nodes, embed_table,
                W_enc, b_enc, W_comp, b_comp)


# speedup vs baseline: 1.0258x; 1.0258x over previous
import functools

import jax
import jax.numpy as jnp
from jax import lax
from jax.experimental import pallas as pl
from jax.experimental.pallas import tpu as pltpu
from jax.experimental.pallas import tpu_sc as plsc

VOCAB = 100000
WDIM = 128
MODEL_DIM = 128
HID = MODEL_DIM // 2
B = 256
L = 128
DEPTH = 64
NSTEP = DEPTH - 1
BSZ = 2 * B
SLOTS = 2 * L

NW = 32
TOK = BSZ * L
TOK_PER_W = TOK // NW
CHUNK = 128
NCHUNK = TOK_PER_W // CHUNK

E = 128
G = BSZ // E


def _sc_embed_gather(table, idx):
    mesh = plsc.VectorSubcoreMesh(core_axis_name="c", subcore_axis_name="s")

    @functools.partial(
        pl.kernel,
        out_type=jax.ShapeDtypeStruct((TOK, WDIM), jnp.float32),
        mesh=mesh,
        scratch_types=[
            pltpu.VMEM((NCHUNK, CHUNK), jnp.int32),
            pltpu.VMEM((CHUNK, WDIM), jnp.float32),
            pltpu.VMEM((CHUNK, WDIM), jnp.float32),
            pltpu.SemaphoreType.DMA,
            pltpu.SemaphoreType.DMA,
        ],
    )
    def body(table_hbm, idx_hbm, out_hbm, idx_v, rows0, rows1, sem0, sem1):
        wid = lax.axis_index("s") * 2 + lax.axis_index("c")
        base = wid * TOK_PER_W
        pltpu.sync_copy(idx_hbm.at[wid], idx_v)
        bufs = (rows0, rows1)
        sems = (sem0, sem1)
        copies = [None, None]
        copies[0] = pltpu.async_copy(table_hbm.at[idx_v.at[0]], rows0, sem0)
        for j in range(NCHUNK):
            p = j % 2
            copies[p].wait()
            if j + 1 < NCHUNK:
                q = (j + 1) % 2
                copies[q] = pltpu.async_copy(
                    table_hbm.at[idx_v.at[j + 1]], bufs[q], sems[q])
            pltpu.sync_copy(bufs[p], out_hbm.at[pl.ds(base + j * CHUNK, CHUNK)])

    return body(table, idx)


E2 = E // 2


def _tc_tree_body(emb_ref, l_ref, r_ref, w_ref, wsh_ref, we_ref, be_ref,
                  wc_ref, bc_ref, out_ref, buf_ref,
                  lfa_ref, rta_ref, sta_ref, lfb_ref, rtb_ref, stb_ref):
    proj = jnp.dot(emb_ref[...], we_ref[...],
                   preferred_element_type=jnp.float32) + be_ref[...]
    buf_ref[:, :L, :] = proj.reshape(E, L, MODEL_DIM)
    buf_ref[:, L:, :] = jnp.zeros((E, SLOTS - L, MODEL_DIM), jnp.float32)
    stb_ref[...] = jnp.zeros((E2, MODEL_DIM), jnp.float32)

    def lstm(lf, rt):
        hl, cl = lf[:, :HID], lf[:, HID:]
        hr, cr = rt[:, :HID], rt[:, HID:]
        gates = (jnp.dot(hl, wc_ref[:HID, :], preferred_element_type=jnp.float32)
                 + jnp.dot(hr, wc_ref[HID:, :], preferred_element_type=jnp.float32)
                 + bc_ref[...])
        i_g = gates[:, 0 * HID:1 * HID]
        f_l = gates[:, 1 * HID:2 * HID]
        f_r = gates[:, 2 * HID:3 * HID]
        o_g = gates[:, 3 * HID:4 * HID]
        g_g = gates[:, 4 * HID:5 * HID]
        c = (jax.nn.sigmoid(f_l) * cl + jax.nn.sigmoid(f_r) * cr
             + jax.nn.sigmoid(i_g) * jnp.tanh(g_g))
        h = jax.nn.sigmoid(o_g) * jnp.tanh(c)
        return jnp.concatenate([h, c], axis=1)

    for e in range(E2):
        lfa_ref[e, :] = buf_ref[e, l_ref[0, e], :]
        rta_ref[e, :] = buf_ref[e, r_ref[0, e], :]

    def step(i, carry):
        for e in range(E2):
            buf_ref[E2 + e, wsh_ref[i, E2 + e], :] = stb_ref[e, :]
        for e in range(E2):
            lfb_ref[e, :] = buf_ref[E2 + e, l_ref[i, E2 + e], :]
            rtb_ref[e, :] = buf_ref[E2 + e, r_ref[i, E2 + e], :]
        sta_ref[...] = lstm(lfa_ref[...], rta_ref[...])
        stb_ref[...] = lstm(lfb_ref[...], rtb_ref[...])
        for e in range(E2):
            buf_ref[e, w_ref[i, e], :] = sta_ref[e, :]
        for e in range(E2):
            lfa_ref[e, :] = buf_ref[e, l_ref[i + 1, e], :]
            rta_ref[e, :] = buf_ref[e, r_ref[i + 1, e], :]
        return carry

    lax.fori_loop(0, NSTEP, step, 0)
    out_ref[:E2, :] = sta_ref[...]
    out_ref[E2:, :] = stb_ref[...]


def _tc_tree(emb, l_adj, r_adj, w_adj, w_shift, W_enc, b_enc, W_comp, b_comp):
    return pl.pallas_call(
        _tc_tree_body,
        grid=(G,),
        in_specs=[
            pl.BlockSpec((E * L, WDIM), lambda g: (g, 0)),
            pl.BlockSpec((DEPTH, E), lambda g: (0, g), memory_space=pltpu.SMEM),
            pl.BlockSpec((DEPTH, E), lambda g: (0, g), memory_space=pltpu.SMEM),
            pl.BlockSpec((DEPTH, E), lambda g: (0, g), memory_space=pltpu.SMEM),
            pl.BlockSpec((DEPTH, E), lambda g: (0, g), memory_space=pltpu.SMEM),
            pl.BlockSpec((WDIM, MODEL_DIM), lambda g: (0, 0)),
            pl.BlockSpec((1, MODEL_DIM), lambda g: (0, 0)),
            pl.BlockSpec((2 * HID, 5 * HID), lambda g: (0, 0)),
            pl.BlockSpec((1, 5 * HID), lambda g: (0, 0)),
        ],
        out_specs=pl.BlockSpec((E, MODEL_DIM), lambda g: (g, 0)),
        out_shape=jax.ShapeDtypeStruct((BSZ, MODEL_DIM), jnp.float32),
        scratch_shapes=[
            pltpu.VMEM((E, SLOTS, MODEL_DIM), jnp.float32),
            pltpu.VMEM((E2, MODEL_DIM), jnp.float32),
            pltpu.VMEM((E2, MODEL_DIM), jnp.float32),
            pltpu.VMEM((E2, MODEL_DIM), jnp.float32),
            pltpu.VMEM((E2, MODEL_DIM), jnp.float32),
            pltpu.VMEM((E2, MODEL_DIM), jnp.float32),
            pltpu.VMEM((E2, MODEL_DIM), jnp.float32),
        ],
    )(emb, l_adj, r_adj, w_adj, w_shift, W_enc, b_enc, W_comp, b_comp)


@jax.jit
def _run(sentences, left_nodes, right_nodes, write_nodes, embed_table,
         W_enc, b_enc, W_comp, b_comp):
    x = jnp.concatenate([sentences[:, :, 0], sentences[:, :, 1]], axis=0)
    idx = x.reshape(NW, NCHUNK, CHUNK)

    l = jnp.concatenate([left_nodes[:, :, 0], left_nodes[:, :, 1]], axis=0)
    r = jnp.concatenate([right_nodes[:, :, 0], right_nodes[:, :, 1]], axis=0)
    w = jnp.concatenate([write_nodes[:, :, 0], write_nodes[:, :, 1]], axis=0)
    l = l - (l >= 200).astype(l.dtype) * (200 - DEPTH)
    r = r - (r >= 200).astype(r.dtype) * (200 - DEPTH)
    w = w - (w >= 201).astype(w.dtype) * (201 - DEPTH)
    w = w + (w <= 0).astype(w.dtype) * (2 * DEPTH)
    w_shift = jnp.concatenate(
        [jnp.full((BSZ, 1), SLOTS - 1, w.dtype), w[:, :-1]], axis=1)
    l, r, w, w_shift = l.T, r.T, w.T, w_shift.T

    emb = _sc_embed_gather(embed_table, idx)
    return _tc_tree(emb, l, r, w, w_shift, W_enc, b_enc.reshape(1, MODEL_DIM),
                    W_comp, b_comp.reshape(1, 5 * HID))


def kernel(sentences, left_nodes, right_nodes, write_nodes, embed_table,
           W_enc, b_enc, W_comp, b_comp):
    return _run(sentences, left_nodes, right_nodes, write_nodes, embed_table,
                W_enc, b_enc, W_comp, b_comp)
